# baseline (speedup 1.0000x reference)
import jax
import jax.numpy as jnp
from jax.experimental import pallas as pl

NUM_EXPERTS = 64
TOP_K = 8
BT = 512


def _m(v):
    return jnp.where(v >= 0, v, jnp.int32(-(2**31)) - v)


def _router_body(x_ref, w_ref, logits_ref, idx_ref, wts_ref):
    x = x_ref[...]
    w = w_ref[...]
    logits = jax.lax.dot_general(
        x, w, (((1,), (1,)), ((), ())), preferred_element_type=jnp.float32
    )
    logits_ref[...] = logits

    col = jax.lax.broadcasted_iota(jnp.int32, (BT, NUM_EXPERTS), 1)
    b = jax.lax.bitcast_convert_type(logits, jnp.int32)
    key = (_m(b) & jnp.int32(~63)) | (jnp.int32(NUM_EXPERTS - 1) - col)
    cur = jax.lax.bitcast_convert_type(_m(key), jnp.float32)

    neg_inf = jnp.float32(-jnp.inf)
    ms = []
    for _ in range(TOP_K):
        m = jnp.max(cur, axis=1, keepdims=True)
        ms.append(m)
        cur = jnp.where(cur == m, neg_inf, cur)
    fm = jnp.concatenate(ms, axis=1)

    kk = _m(jax.lax.bitcast_convert_type(fm, jnp.int32))
    idx0 = jnp.int32(NUM_EXPERTS - 1) - (kk & jnp.int32(63))
    ev0 = jnp.take_along_axis(logits, idx0, axis=1)

    ev = ev0
    ix = idx0

    def _shl(a):
        return jnp.concatenate([a[:, 1:], a[:, TOP_K - 1:]], axis=1)

    def _shr(a):
        return jnp.concatenate([jnp.zeros_like(a[:, :1]), a[:, : TOP_K - 1]], axis=1)

    posin = jax.lax.broadcasted_iota(jnp.int32, (BT, TOP_K), 1)
    for parity in (0, 1, 0):
        ev_r = _shl(ev)
        ix_r = _shl(ix)
        beat = (ev_r > ev) | ((ev_r == ev) & (ix_r < ix))
        can = (posin % 2 == parity) & (posin < TOP_K - 1)
        swap = jnp.where(beat & can, jnp.int32(1), jnp.int32(0))
        swap_l = _shr(swap)
        ev_l = _shr(ev)
        ix_l = _shr(ix)
        ev = jnp.where(swap == 1, ev_r, jnp.where(swap_l == 1, ev_l, ev))
        ix = jnp.where(swap == 1, ix_r, jnp.where(swap_l == 1, ix_l, ix))

    idx_ref[...] = ix
    e = jnp.exp(ev - ev[:, 0:1])
    wts_ref[...] = e / jnp.sum(e, axis=1, keepdims=True)


def kernel(hidden_states, W_gate):
    if hidden_states.ndim == 3:
        hidden_states = hidden_states.reshape(-1, hidden_states.shape[-1])
    T, H = hidden_states.shape
    E = W_gate.shape[0]
    grid = (T // BT,)
    logits, idx, wts = pl.pallas_call(
        _router_body,
        grid=grid,
        in_specs=[
            pl.BlockSpec((BT, H), lambda i: (i, 0)),
            pl.BlockSpec((E, H), lambda i: (0, 0)),
        ],
        out_specs=[
            pl.BlockSpec((BT, E), lambda i: (i, 0)),
            pl.BlockSpec((BT, TOP_K), lambda i: (i, 0)),
            pl.BlockSpec((BT, TOP_K), lambda i: (i, 0)),
        ],
        out_shape=[
            jax.ShapeDtypeStruct((T, E), jnp.float32),
            jax.ShapeDtypeStruct((T, TOP_K), jnp.int32),
            jax.ShapeDtypeStruct((T, TOP_K), jnp.float32),
        ],
    )(hidden_states, W_gate)
    return (logits, idx, wts)


# speedup vs baseline: 1.0212x; 1.0212x over previous
import jax
import jax.numpy as jnp
from jax.experimental import pallas as pl
from jax.experimental.pallas import tpu as pltpu

NUM_EXPERTS = 64
TOP_K = 8
BT = 512


def _m(v):
    return jnp.where(v >= 0, v, jnp.int32(-(2**31)) - v)


def _topk8(logits, idx_ref, wts_ref):
    col = jax.lax.broadcasted_iota(jnp.int32, (BT, NUM_EXPERTS), 1)
    b = jax.lax.bitcast_convert_type(logits, jnp.int32)
    key = (_m(b) & jnp.int32(~63)) | (jnp.int32(NUM_EXPERTS - 1) - col)
    cur = jax.lax.bitcast_convert_type(_m(key), jnp.float32)

    neg_inf = jnp.float32(-jnp.inf)
    ms = []
    for _ in range(TOP_K):
        m = jnp.max(cur, axis=1, keepdims=True)
        ms.append(m)
        cur = jnp.where(cur == m, neg_inf, cur)
    fm = jnp.concatenate(ms, axis=1)

    kk = _m(jax.lax.bitcast_convert_type(fm, jnp.int32))
    idx0 = jnp.int32(NUM_EXPERTS - 1) - (kk & jnp.int32(63))
    ev0 = jnp.take_along_axis(logits, idx0, axis=1)

    ev = ev0
    ix = idx0

    def _shl(a):
        return jnp.concatenate([a[:, 1:], a[:, TOP_K - 1:]], axis=1)

    def _shr(a):
        return jnp.concatenate([jnp.zeros_like(a[:, :1]), a[:, : TOP_K - 1]], axis=1)

    posin = jax.lax.broadcasted_iota(jnp.int32, (BT, TOP_K), 1)
    for parity in (0, 1, 0):
        ev_r = _shl(ev)
        ix_r = _shl(ix)
        beat = (ev_r > ev) | ((ev_r == ev) & (ix_r < ix))
        can = (posin % 2 == parity) & (posin < TOP_K - 1)
        swap = jnp.where(beat & can, jnp.int32(1), jnp.int32(0))
        swap_l = _shr(swap)
        ev_l = _shr(ev)
        ix_l = _shr(ix)
        ev = jnp.where(swap == 1, ev_r, jnp.where(swap_l == 1, ev_l, ev))
        ix = jnp.where(swap == 1, ix_r, jnp.where(swap_l == 1, ix_l, ix))

    idx_ref[...] = ix
    e = jnp.exp(ev - ev[:, 0:1])
    wts_ref[...] = e / jnp.sum(e, axis=1, keepdims=True)


def _make_body(num_blocks):
    def _router_body(x_ref, w_ref, logits_ref, idx_ref, wts_ref, prev_ref):
        i = pl.program_id(0)

        @pl.when(i > 0)
        def _do_topk():
            _topk8(prev_ref[...], idx_ref, wts_ref)

        @pl.when(i < num_blocks)
        def _do_matmul():
            logits = jax.lax.dot_general(
                x_ref[...], w_ref[...], (((1,), (1,)), ((), ())),
                preferred_element_type=jnp.float32,
            )
            logits_ref[...] = logits
            prev_ref[...] = logits

    return _router_body


def kernel(hidden_states, W_gate):
    if hidden_states.ndim == 3:
        hidden_states = hidden_states.reshape(-1, hidden_states.shape[-1])
    T, H = hidden_states.shape
    E = W_gate.shape[0]
    nb = T // BT
    last = nb - 1
    logits, idx, wts = pl.pallas_call(
        _make_body(nb),
        grid=(nb + 1,),
        in_specs=[
            pl.BlockSpec((BT, H), lambda i: (jnp.minimum(i, last), 0)),
            pl.BlockSpec((E, H), lambda i: (0, 0)),
        ],
        out_specs=[
            pl.BlockSpec((BT, E), lambda i: (jnp.minimum(i, last), 0)),
            pl.BlockSpec((BT, TOP_K), lambda i: (jnp.maximum(i - 1, 0), 0)),
            pl.BlockSpec((BT, TOP_K), lambda i: (jnp.maximum(i - 1, 0), 0)),
        ],
        out_shape=[
            jax.ShapeDtypeStruct((T, E), jnp.float32),
            jax.ShapeDtypeStruct((T, TOP_K), jnp.int32),
            jax.ShapeDtypeStruct((T, TOP_K), jnp.float32),
        ],
        scratch_shapes=[pltpu.VMEM((BT, NUM_EXPERTS), jnp.float32)],
    )(hidden_states, W_gate)
    return (logits, idx, wts)


# speedup vs baseline: 1.1614x; 1.1373x over previous
import jax
import jax.numpy as jnp
from jax.experimental import pallas as pl
from jax.experimental.pallas import tpu as pltpu

NUM_EXPERTS = 64
TOP_K = 8
BT = 512


def _m(v):
    return jnp.where(v >= 0, v, jnp.int32(-(2**31)) - v)


def _topk8(logits, idx_ref, wts_ref):
    col = jax.lax.broadcasted_iota(jnp.int32, (BT, NUM_EXPERTS), 1)
    b = jax.lax.bitcast_convert_type(logits, jnp.int32)
    key = (_m(b) & jnp.int32(~63)) | (jnp.int32(NUM_EXPERTS - 1) - col)
    cur = jax.lax.bitcast_convert_type(_m(key), jnp.float32)

    neg_inf = jnp.float32(-jnp.inf)
    ms = []
    for _ in range(TOP_K):
        m = jnp.max(cur, axis=1, keepdims=True)
        ms.append(m)
        cur = jnp.where(cur == m, neg_inf, cur)
    fm = jnp.concatenate(ms, axis=1)

    kk = _m(jax.lax.bitcast_convert_type(fm, jnp.int32))
    idx0 = jnp.int32(NUM_EXPERTS - 1) - (kk & jnp.int32(63))
    ev0 = jnp.take_along_axis(logits, idx0, axis=1)

    ev = ev0
    ix = idx0

    def _shl(a):
        return jnp.concatenate([a[:, 1:], a[:, TOP_K - 1:]], axis=1)

    def _shr(a):
        return jnp.concatenate([jnp.zeros_like(a[:, :1]), a[:, : TOP_K - 1]], axis=1)

    posin = jax.lax.broadcasted_iota(jnp.int32, (BT, TOP_K), 1)
    for parity in (0, 1, 0):
        ev_r = _shl(ev)
        ix_r = _shl(ix)
        beat = (ev_r > ev) | ((ev_r == ev) & (ix_r < ix))
        can = (posin % 2 == parity) & (posin < TOP_K - 1)
        swap = jnp.where(beat & can, jnp.int32(1), jnp.int32(0))
        swap_l = _shr(swap)
        ev_l = _shr(ev)
        ix_l = _shr(ix)
        ev = jnp.where(swap == 1, ev_r, jnp.where(swap_l == 1, ev_l, ev))
        ix = jnp.where(swap == 1, ix_r, jnp.where(swap_l == 1, ix_l, ix))

    idx_ref[...] = ix
    e = jnp.exp(ev - ev[:, 0:1])
    wts_ref[...] = e / jnp.sum(e, axis=1, keepdims=True)


def _make_body(num_blocks):
    def _router_body(x_ref, w_ref, logits_ref, idx_ref, wts_ref, prev_ref):
        _topk8(prev_ref[...], idx_ref, wts_ref)
        logits = jax.lax.dot_general(
            x_ref[...], w_ref[...], (((1,), (1,)), ((), ())),
            preferred_element_type=jnp.float32,
        )
        logits_ref[...] = logits
        prev_ref[...] = logits

    return _router_body


def kernel(hidden_states, W_gate):
    if hidden_states.ndim == 3:
        hidden_states = hidden_states.reshape(-1, hidden_states.shape[-1])
    T, H = hidden_states.shape
    E = W_gate.shape[0]
    nb = T // BT
    last = nb - 1
    logits, idx, wts = pl.pallas_call(
        _make_body(nb),
        grid=(nb + 1,),
        in_specs=[
            pl.BlockSpec((BT, H), lambda i: (jnp.minimum(i, last), 0)),
            pl.BlockSpec((E, H), lambda i: (0, 0)),
        ],
        out_specs=[
            pl.BlockSpec((BT, E), lambda i: (jnp.minimum(i, last), 0)),
            pl.BlockSpec((BT, TOP_K), lambda i: (jnp.maximum(i - 1, 0), 0)),
            pl.BlockSpec((BT, TOP_K), lambda i: (jnp.maximum(i - 1, 0), 0)),
        ],
        out_shape=[
            jax.ShapeDtypeStruct((T, E), jnp.float32),
            jax.ShapeDtypeStruct((T, TOP_K), jnp.int32),
            jax.ShapeDtypeStruct((T, TOP_K), jnp.float32),
        ],
        scratch_shapes=[pltpu.VMEM((BT, NUM_EXPERTS), jnp.float32)],
    )(hidden_states, W_gate)
    return (logits, idx, wts)


# speedup vs baseline: 1.6251x; 1.3992x over previous
import jax
import jax.numpy as jnp
from jax.experimental import pallas as pl
from jax.experimental.pallas import tpu as pltpu

NUM_EXPERTS = 64
TOP_K = 8
BT = 512


def _m(v):
    return jnp.where(v >= 0, v, jnp.int32(-(2**31)) - v)


def _rot_rows(a, s):
    return jnp.concatenate([a[s:], a[:s]], axis=0)


def _max8(a):
    a = jnp.maximum(a[:32], a[32:])
    a = jnp.maximum(a[:16], a[16:])
    a = jnp.maximum(a[:8], a[8:])
    a = jnp.maximum(a, _rot_rows(a, 4))
    a = jnp.maximum(a, _rot_rows(a, 2))
    a = jnp.maximum(a, _rot_rows(a, 1))
    return a


def _topk8_t(logits_t, idx_ref, wts_ref):
    row = jax.lax.broadcasted_iota(jnp.int32, (NUM_EXPERTS, BT), 0)
    row8 = jax.lax.broadcasted_iota(jnp.int32, (TOP_K, BT), 0)
    b = jax.lax.bitcast_convert_type(logits_t, jnp.int32)
    key = (_m(b) & jnp.int32(~63)) | (jnp.int32(NUM_EXPERTS - 1) - row)
    cur = jax.lax.bitcast_convert_type(_m(key), jnp.float32)

    neg_inf = jnp.float32(-jnp.inf)
    kt = jnp.zeros((TOP_K, BT), jnp.float32)
    evt = jnp.zeros((TOP_K, BT), jnp.float32)
    for k in range(TOP_K):
        mk8 = _max8(cur)
        mk = jnp.concatenate([mk8] * (NUM_EXPERTS // TOP_K), axis=0)
        onehot = cur == mk
        ev8 = _max8(jnp.where(onehot, logits_t, neg_inf))
        sel = row8 == k
        kt = jnp.where(sel, mk8, kt)
        evt = jnp.where(sel, ev8, evt)
        cur = jnp.where(onehot, neg_inf, cur)

    kk = _m(jax.lax.bitcast_convert_type(kt, jnp.int32))
    ixt = jnp.int32(NUM_EXPERTS - 1) - (kk & jnp.int32(63))

    for parity in (0, 1):
        ev_r = _rot_rows(evt, 1)
        ix_r = _rot_rows(ixt, 1)
        beat = (ev_r > evt) | ((ev_r == evt) & (ix_r < ixt))
        can = (row8 % 2 == parity) & (row8 < TOP_K - 1)
        swap = jnp.where(beat & can, jnp.int32(1), jnp.int32(0))
        swap_l = _rot_rows(swap, TOP_K - 1)
        ev_l = _rot_rows(evt, TOP_K - 1)
        ix_l = _rot_rows(ixt, TOP_K - 1)
        swap_l = jnp.where(row8 == 0, 0, swap_l)
        evt = jnp.where(swap == 1, ev_r, jnp.where(swap_l == 1, ev_l, evt))
        ixt = jnp.where(swap == 1, ix_r, jnp.where(swap_l == 1, ix_l, ixt))

    idx_ref[...] = ixt
    mx = jnp.concatenate([evt[0:1]] * TOP_K, axis=0)
    e = jnp.exp(evt - mx)
    s = e
    s = s + _rot_rows(s, 4)
    s = s + _rot_rows(s, 2)
    s = s + _rot_rows(s, 1)
    wts_ref[...] = e / s


def _router_body(x_ref, w_ref, logits_ref, idx_ref, wts_ref, prev_ref):
    _topk8_t(prev_ref[...], idx_ref, wts_ref)
    logits_t = jax.lax.dot_general(
        w_ref[...], x_ref[...], (((1,), (1,)), ((), ())),
        preferred_element_type=jnp.float32,
    )
    logits_ref[...] = logits_t
    prev_ref[...] = logits_t


def kernel(hidden_states, W_gate):
    if hidden_states.ndim == 3:
        hidden_states = hidden_states.reshape(-1, hidden_states.shape[-1])
    T, H = hidden_states.shape
    E = W_gate.shape[0]
    nb = T // BT
    last = nb - 1
    logits_t, idx_t, wts_t = pl.pallas_call(
        _router_body,
        grid=(nb + 1,),
        in_specs=[
            pl.BlockSpec((BT, H), lambda i: (jnp.minimum(i, last), 0)),
            pl.BlockSpec((E, H), lambda i: (0, 0)),
        ],
        out_specs=[
            pl.BlockSpec((E, BT), lambda i: (0, jnp.minimum(i, last))),
            pl.BlockSpec((TOP_K, BT), lambda i: (0, jnp.maximum(i - 1, 0))),
            pl.BlockSpec((TOP_K, BT), lambda i: (0, jnp.maximum(i - 1, 0))),
        ],
        out_shape=[
            jax.ShapeDtypeStruct((E, T), jnp.float32),
            jax.ShapeDtypeStruct((TOP_K, T), jnp.int32),
            jax.ShapeDtypeStruct((TOP_K, T), jnp.float32),
        ],
        scratch_shapes=[pltpu.VMEM((E, BT), jnp.float32)],
    )(hidden_states, W_gate)
    return (logits_t.T, idx_t.T, wts_t.T)
